# baseline (speedup 1.0000x reference)
import functools

import jax
import jax.numpy as jnp
from jax import lax
from jax.experimental import pallas as pl
from jax.experimental.pallas import tpu as pltpu
from jax.experimental.pallas import tpu_sc as plsc

NN = 10000
NE = 320000
DD = 128
HALF = 64
NC = 2
NS = 16
NW = NC * NS
EPW = NE // NW
BB = 80
NBATCH = EPW // BB
NCHUNK = 5
CB = NBATCH // NCHUNK
NTRI = (CB - 1) // 3
RPT = 640


def _mm_body(x_ref, wt_ref, b_ref, tdb0_ref, te0_ref, tdb1_ref, te1_ref,
             ax_ref):
    xw = jnp.dot(x_ref[...], wt_ref[...],
                 preferred_element_type=jnp.float32) + b_ref[...]
    tdb0_ref[...] = xw[:, 0:128]
    te0_ref[...] = xw[:, 128:192]
    tdb1_ref[...] = xw[:, 192:320]
    te1_ref[...] = xw[:, 320:384]
    ax_ref[...] = xw[:, 384:512]


_mm = pl.pallas_call(
    _mm_body,
    out_shape=[
        jax.ShapeDtypeStruct((NN, 2 * HALF), jnp.float32),
        jax.ShapeDtypeStruct((NN, HALF), jnp.float32),
        jax.ShapeDtypeStruct((NN, 2 * HALF), jnp.float32),
        jax.ShapeDtypeStruct((NN, HALF), jnp.float32),
        jax.ShapeDtypeStruct((NN, DD), jnp.float32),
    ],
)


def _compute_rows(dbv, ev):
    def row_body(r2, c2):
        for u in range(2):
            r = 2 * r2 + u
            for c in range(HALF // 16):
                d = dbv[r, pl.ds(c * 16, 16)]
                e_ = ev[r, pl.ds(c * 16, 16)]
                bx = dbv[r, pl.ds(HALF + c * 16, 16)]
                s = 1.0 / (1.0 + jnp.exp(-(d + e_)))
                dbv[r, pl.ds(c * 16, 16)] = s * bx
                dbv[r, pl.ds(HALF + c * 16, 16)] = s
        return c2

    lax.fori_loop(0, BB // 2, row_body, 0)


def _sc_body(tdb0, te0, tdb1, te1, src2, dst2, zz, nd0_o, nd1_o,
             siv, div, dbv0, ev0, dbv1, ev1, dbv2, ev2, ndsh,
             sga0, sgb0, sga1, sgb1, sga2, sgb2, ssc0, ssc1, ssc2):
    cid = lax.axis_index("c")
    sid = lax.axis_index("s")
    wid = sid * NC + cid
    rbase = pl.multiple_of(sid * RPT, 8)

    for h in range(2):
        tdb = (tdb0, tdb1)[h]
        te = (te0, te1)[h]
        nd_o = (nd0_o, nd1_o)[h]

        @pl.when(sid < NS - 1)
        def _():
            pltpu.sync_copy(zz.at[pl.ds(rbase, RPT)],
                            ndsh.at[pl.ds(rbase, RPT)])

        @pl.when(sid == NS - 1)
        def _():
            pltpu.sync_copy(zz.at[pl.ds(9600, 400)], ndsh.at[pl.ds(9600, 400)])

        plsc.subcore_barrier()

        dbvs = (dbv0, dbv1, dbv2)
        evs = (ev0, ev1, ev2)
        sgas = (sga0, sga1, sga2)
        sgbs = (sgb0, sgb1, sgb2)
        sscs = (ssc0, ssc1, ssc2)

        for ci in range(NCHUNK):
            pltpu.sync_copy(src2.at[wid * NCHUNK + ci], siv)
            pltpu.sync_copy(dst2.at[wid * NCHUNK + ci], div)

            for p in range(3):
                pltpu.async_copy(tdb.at[siv.at[p]], dbvs[p], sgas[p])
                pltpu.async_copy(te.at[div.at[p]], evs[p], sgbs[p])

            def tri_body(i, carry):
                b0 = 3 * i
                for p in range(3):
                    b = b0 + p
                    pv = (p - 1) % 3
                    pltpu.make_async_copy(tdb.at[siv.at[b]], dbvs[p],
                                          sgas[p]).wait()
                    pltpu.make_async_copy(te.at[div.at[b]], evs[p],
                                          sgbs[p]).wait()
                    _compute_rows(dbvs[p], evs[p])
                    pltpu.async_copy(dbvs[p], ndsh.at[div.at[b]], sscs[p],
                                     add=True)
                    fire = (b0 >= 1) if p == 0 else (b0 <= CB - 3 - p)

                    @pl.when(fire)
                    def _():
                        pltpu.make_async_copy(dbvs[pv],
                                              ndsh.at[div.at[b - 1]],
                                              sscs[pv]).wait()
                        pltpu.async_copy(tdb.at[siv.at[b + 2]], dbvs[pv],
                                         sgas[pv])
                        pltpu.async_copy(te.at[div.at[b + 2]], evs[pv],
                                         sgbs[pv])
                return carry

            lax.fori_loop(0, NTRI, tri_body, 0)

            last = CB - 1
            lp = last % 3
            pltpu.make_async_copy(tdb.at[siv.at[last]], dbvs[lp],
                                  sgas[lp]).wait()
            pltpu.make_async_copy(te.at[div.at[last]], evs[lp],
                                  sgbs[lp]).wait()
            _compute_rows(dbvs[lp], evs[lp])
            pltpu.async_copy(dbvs[lp], ndsh.at[div.at[last]], sscs[lp],
                             add=True)
            for c in (last - 2, last - 1, last):
                pltpu.make_async_copy(dbvs[c % 3], ndsh.at[div.at[c]],
                                      sscs[c % 3]).wait()

        plsc.subcore_barrier()

        @pl.when(sid < NS - 1)
        def _():
            ob = pl.multiple_of(cid * NN + rbase, 8)
            pltpu.sync_copy(ndsh.at[pl.ds(rbase, RPT)], nd_o.at[pl.ds(ob, RPT)])

        @pl.when(sid == NS - 1)
        def _():
            ob = pl.multiple_of(cid * NN + 9600, 8)
            pltpu.sync_copy(ndsh.at[pl.ds(9600, 400)], nd_o.at[pl.ds(ob, 400)])

        if h == 0:
            plsc.subcore_barrier()


_sc_edge = functools.partial(
    pl.kernel,
    mesh=plsc.VectorSubcoreMesh(core_axis_name="c", subcore_axis_name="s"),
    compiler_params=pltpu.CompilerParams(use_tc_tiling_on_sc=False),
    out_type=[
        jax.ShapeDtypeStruct((NC * NN, 2 * HALF), jnp.float32),
        jax.ShapeDtypeStruct((NC * NN, 2 * HALF), jnp.float32),
    ],
    scratch_types=[
        pltpu.VMEM((CB, BB), jnp.int32),
        pltpu.VMEM((CB, BB), jnp.int32),
        pltpu.VMEM((BB, 2 * HALF), jnp.float32),
        pltpu.VMEM((BB, HALF), jnp.float32),
        pltpu.VMEM((BB, 2 * HALF), jnp.float32),
        pltpu.VMEM((BB, HALF), jnp.float32),
        pltpu.VMEM((BB, 2 * HALF), jnp.float32),
        pltpu.VMEM((BB, HALF), jnp.float32),
        pltpu.VMEM_SHARED((NN, 2 * HALF), jnp.float32),
        pltpu.SemaphoreType.DMA,
        pltpu.SemaphoreType.DMA,
        pltpu.SemaphoreType.DMA,
        pltpu.SemaphoreType.DMA,
        pltpu.SemaphoreType.DMA,
        pltpu.SemaphoreType.DMA,
        pltpu.SemaphoreType.DMA,
        pltpu.SemaphoreType.DMA,
        pltpu.SemaphoreType.DMA,
    ],
)(_sc_body)


def _fin_body(x_ref, ax_ref, nd0_ref, nd1_ref, g_ref, b_ref, o_ref):
    for j, nd_ref in enumerate((nd0_ref, nd1_ref)):
        cs = slice(j * HALF, (j + 1) * HALF)
        x = x_ref[:, cs]
        nd = nd_ref[0:NN, :] + nd_ref[NN:2 * NN, :]
        num = nd[:, 0:HALF]
        den = nd[:, HALF:2 * HALF]
        rowmask = jnp.max(den, axis=1, keepdims=True) > 0
        h = ax_ref[:, cs] + num / jnp.where(den > 0, den, 1.0)
        h = jnp.where(rowmask, h, x)
        h = h * (1.0 / NN)
        mean = jnp.mean(h, axis=0, keepdims=True)
        var = jnp.mean((h - mean) ** 2, axis=0, keepdims=True)
        hn = (h - mean) * lax.rsqrt(var + 1e-5) * g_ref[:, cs] + b_ref[:, cs]
        o_ref[:, cs] = x + jnp.maximum(hn, 0.0)


_fin = pl.pallas_call(
    _fin_body,
    out_shape=jax.ShapeDtypeStruct((NN, DD), jnp.float32),
)


def kernel(X, edge_index, A_w, A_b, B_w, B_b, D_w, D_b, E_w, E_b,
           bn_gamma, bn_beta):
    src2 = edge_index[0].astype(jnp.int32).reshape(NW * NCHUNK, CB, BB)
    dst2 = edge_index[1].astype(jnp.int32).reshape(NW * NCHUNK, CB, BB)
    W_all = jnp.concatenate(
        [D_w[:HALF], B_w[:HALF], E_w[:HALF],
         D_w[HALF:], B_w[HALF:], E_w[HALF:], A_w], axis=0)
    b_all = jnp.concatenate(
        [D_b[:HALF], B_b[:HALF], E_b[:HALF],
         D_b[HALF:], B_b[HALF:], E_b[HALF:], A_b]).reshape(1, -1)
    tdb0, te0, tdb1, te1, ax = _mm(X, W_all.T, b_all)
    zz = jnp.zeros((NN, 2 * HALF), jnp.float32)
    nd0, nd1 = _sc_edge(tdb0, te0, tdb1, te1, src2, dst2, zz)
    return _fin(X, ax, nd0, nd1,
                bn_gamma.reshape(1, -1), bn_beta.reshape(1, -1))


# speedup vs baseline: 1.0041x; 1.0041x over previous
import functools

import jax
import jax.numpy as jnp
from jax import lax
from jax.experimental import pallas as pl
from jax.experimental.pallas import tpu as pltpu
from jax.experimental.pallas import tpu_sc as plsc

NN = 10000
NE = 320000
DD = 128
HALF = 64
NC = 2
NS = 16
NW = NC * NS
EPW = NE // NW
BB = 80
NBATCH = EPW // BB
NCHUNK = 5
CB = NBATCH // NCHUNK
NTRI = (CB - 1) // 3
RPT = 640


def _mm_body(x_ref, wt_ref, b_ref, tdb0_ref, te0_ref, tdb1_ref, te1_ref,
             ax_ref):
    xw = jnp.dot(x_ref[...], wt_ref[...],
                 preferred_element_type=jnp.float32) + b_ref[...]
    tdb0_ref[...] = xw[:, 0:128]
    te0_ref[...] = xw[:, 128:192]
    tdb1_ref[...] = xw[:, 192:320]
    te1_ref[...] = xw[:, 320:384]
    ax_ref[...] = xw[:, 384:512]


_mm = pl.pallas_call(
    _mm_body,
    out_shape=[
        jax.ShapeDtypeStruct((NN, 2 * HALF), jnp.float32),
        jax.ShapeDtypeStruct((NN, HALF), jnp.float32),
        jax.ShapeDtypeStruct((NN, 2 * HALF), jnp.float32),
        jax.ShapeDtypeStruct((NN, HALF), jnp.float32),
        jax.ShapeDtypeStruct((NN, DD), jnp.float32),
    ],
)


def _compute_rows(dbv, ev):
    def row_body(r, c2):
        for c in range(HALF // 16):
            d = dbv[r, pl.ds(c * 16, 16)]
            e_ = ev[r, pl.ds(c * 16, 16)]
            bx = dbv[r, pl.ds(HALF + c * 16, 16)]
            s = 1.0 / (1.0 + jnp.exp(-(d + e_)))
            dbv[r, pl.ds(c * 16, 16)] = s * bx
            dbv[r, pl.ds(HALF + c * 16, 16)] = s
        return c2

    lax.fori_loop(0, BB, row_body, 0)


def _sc_body(tdb0, te0, tdb1, te1, src2, dst2, zz, nd0_o, nd1_o,
             siv, div, dbv0, ev0, dbv1, ev1, dbv2, ev2, ndsh,
             sga0, sgb0, sga1, sgb1, sga2, sgb2, ssc0, ssc1, ssc2):
    cid = lax.axis_index("c")
    sid = lax.axis_index("s")
    wid = sid * NC + cid
    rbase = pl.multiple_of(sid * RPT, 8)

    for h in range(2):
        tdb = (tdb0, tdb1)[h]
        te = (te0, te1)[h]
        nd_o = (nd0_o, nd1_o)[h]

        @pl.when(sid < NS - 1)
        def _():
            pltpu.sync_copy(zz.at[pl.ds(rbase, RPT)],
                            ndsh.at[pl.ds(rbase, RPT)])

        @pl.when(sid == NS - 1)
        def _():
            pltpu.sync_copy(zz.at[pl.ds(9600, 400)], ndsh.at[pl.ds(9600, 400)])

        plsc.subcore_barrier()

        dbvs = (dbv0, dbv1, dbv2)
        evs = (ev0, ev1, ev2)
        sgas = (sga0, sga1, sga2)
        sgbs = (sgb0, sgb1, sgb2)
        sscs = (ssc0, ssc1, ssc2)

        for ci in range(NCHUNK):
            pltpu.sync_copy(src2.at[wid * NCHUNK + ci], siv)
            pltpu.sync_copy(dst2.at[wid * NCHUNK + ci], div)

            for p in range(3):
                pltpu.async_copy(tdb.at[siv.at[p]], dbvs[p], sgas[p])
                pltpu.async_copy(te.at[div.at[p]], evs[p], sgbs[p])

            def tri_body(i, carry):
                b0 = 3 * i
                for p in range(3):
                    b = b0 + p
                    pv = (p - 1) % 3
                    pltpu.make_async_copy(tdb.at[siv.at[b]], dbvs[p],
                                          sgas[p]).wait()
                    pltpu.make_async_copy(te.at[div.at[b]], evs[p],
                                          sgbs[p]).wait()
                    _compute_rows(dbvs[p], evs[p])
                    pltpu.async_copy(dbvs[p], ndsh.at[div.at[b]], sscs[p],
                                     add=True)
                    fire = (b0 >= 1) if p == 0 else (b0 <= CB - 3 - p)

                    @pl.when(fire)
                    def _():
                        pltpu.make_async_copy(dbvs[pv],
                                              ndsh.at[div.at[b - 1]],
                                              sscs[pv]).wait()
                        pltpu.async_copy(tdb.at[siv.at[b + 2]], dbvs[pv],
                                         sgas[pv])
                        pltpu.async_copy(te.at[div.at[b + 2]], evs[pv],
                                         sgbs[pv])
                return carry

            lax.fori_loop(0, NTRI, tri_body, 0)

            last = CB - 1
            lp = last % 3
            pltpu.make_async_copy(tdb.at[siv.at[last]], dbvs[lp],
                                  sgas[lp]).wait()
            pltpu.make_async_copy(te.at[div.at[last]], evs[lp],
                                  sgbs[lp]).wait()
            _compute_rows(dbvs[lp], evs[lp])
            pltpu.async_copy(dbvs[lp], ndsh.at[div.at[last]], sscs[lp],
                             add=True)
            for c in (last - 2, last - 1, last):
                pltpu.make_async_copy(dbvs[c % 3], ndsh.at[div.at[c]],
                                      sscs[c % 3]).wait()

        plsc.subcore_barrier()

        @pl.when(sid < NS - 1)
        def _():
            ob = pl.multiple_of(cid * NN + rbase, 8)
            pltpu.sync_copy(ndsh.at[pl.ds(rbase, RPT)], nd_o.at[pl.ds(ob, RPT)])

        @pl.when(sid == NS - 1)
        def _():
            ob = pl.multiple_of(cid * NN + 9600, 8)
            pltpu.sync_copy(ndsh.at[pl.ds(9600, 400)], nd_o.at[pl.ds(ob, 400)])

        if h == 0:
            plsc.subcore_barrier()


_sc_edge = functools.partial(
    pl.kernel,
    mesh=plsc.VectorSubcoreMesh(core_axis_name="c", subcore_axis_name="s"),
    compiler_params=pltpu.CompilerParams(use_tc_tiling_on_sc=False),
    out_type=[
        jax.ShapeDtypeStruct((NC * NN, 2 * HALF), jnp.float32),
        jax.ShapeDtypeStruct((NC * NN, 2 * HALF), jnp.float32),
    ],
    scratch_types=[
        pltpu.VMEM((CB, BB), jnp.int32),
        pltpu.VMEM((CB, BB), jnp.int32),
        pltpu.VMEM((BB, 2 * HALF), jnp.float32),
        pltpu.VMEM((BB, HALF), jnp.float32),
        pltpu.VMEM((BB, 2 * HALF), jnp.float32),
        pltpu.VMEM((BB, HALF), jnp.float32),
        pltpu.VMEM((BB, 2 * HALF), jnp.float32),
        pltpu.VMEM((BB, HALF), jnp.float32),
        pltpu.VMEM_SHARED((NN, 2 * HALF), jnp.float32),
        pltpu.SemaphoreType.DMA,
        pltpu.SemaphoreType.DMA,
        pltpu.SemaphoreType.DMA,
        pltpu.SemaphoreType.DMA,
        pltpu.SemaphoreType.DMA,
        pltpu.SemaphoreType.DMA,
        pltpu.SemaphoreType.DMA,
        pltpu.SemaphoreType.DMA,
        pltpu.SemaphoreType.DMA,
    ],
)(_sc_body)


def _fin_body(x_ref, ax_ref, nd0_ref, nd1_ref, g_ref, b_ref, o_ref):
    for j, nd_ref in enumerate((nd0_ref, nd1_ref)):
        cs = slice(j * HALF, (j + 1) * HALF)
        x = x_ref[:, cs]
        nd = nd_ref[0:NN, :] + nd_ref[NN:2 * NN, :]
        num = nd[:, 0:HALF]
        den = nd[:, HALF:2 * HALF]
        rowmask = jnp.max(den, axis=1, keepdims=True) > 0
        h = ax_ref[:, cs] + num / jnp.where(den > 0, den, 1.0)
        h = jnp.where(rowmask, h, x)
        h = h * (1.0 / NN)
        mean = jnp.mean(h, axis=0, keepdims=True)
        var = jnp.mean((h - mean) ** 2, axis=0, keepdims=True)
        hn = (h - mean) * lax.rsqrt(var + 1e-5) * g_ref[:, cs] + b_ref[:, cs]
        o_ref[:, cs] = x + jnp.maximum(hn, 0.0)


_fin = pl.pallas_call(
    _fin_body,
    out_shape=jax.ShapeDtypeStruct((NN, DD), jnp.float32),
)


def kernel(X, edge_index, A_w, A_b, B_w, B_b, D_w, D_b, E_w, E_b,
           bn_gamma, bn_beta):
    src2 = edge_index[0].astype(jnp.int32).reshape(NW * NCHUNK, CB, BB)
    dst2 = edge_index[1].astype(jnp.int32).reshape(NW * NCHUNK, CB, BB)
    W_all = jnp.concatenate(
        [D_w[:HALF], B_w[:HALF], E_w[:HALF],
         D_w[HALF:], B_w[HALF:], E_w[HALF:], A_w], axis=0)
    b_all = jnp.concatenate(
        [D_b[:HALF], B_b[:HALF], E_b[:HALF],
         D_b[HALF:], B_b[HALF:], E_b[HALF:], A_b]).reshape(1, -1)
    tdb0, te0, tdb1, te1, ax = _mm(X, W_all.T, b_all)
    zz = jnp.zeros((NN, 2 * HALF), jnp.float32)
    nd0, nd1 = _sc_edge(tdb0, te0, tdb1, te1, src2, dst2, zz)
    return _fin(X, ax, nd0, nd1,
                bn_gamma.reshape(1, -1), bn_beta.reshape(1, -1))
